# baseline (speedup 1.0000x reference)
import functools

import jax
import jax.numpy as jnp
from jax import lax
from jax.experimental import pallas as pl
from jax.experimental.pallas import tpu as pltpu
from jax.experimental.pallas import tpu_sc as plsc

B = 16384
EMB = 128
NUM_LOC = 58
NUM_AGE = 2

NC, NS, L = 2, 16, 16
NW = NC * NS
CPW = B // NW
G = 128
NCHUNK = CPW // G


@functools.lru_cache(maxsize=None)
def _build_sc_gather():
    @functools.partial(
        pl.kernel,
        mesh=plsc.VectorSubcoreMesh(core_axis_name="c", subcore_axis_name="s"),
        compiler_params=pltpu.CompilerParams(
            needs_layout_passes=False,
            skip_device_barrier=True,
            disable_bounds_checks=True,
            disable_semaphore_checks=True,
        ),
        out_type=jax.ShapeDtypeStruct((B, 2 * EMB), jnp.float32),
        scratch_types=[
            pltpu.VMEM((CPW,), jnp.int32),
            pltpu.VMEM((CPW,), jnp.int32),
            pltpu.VMEM((G, EMB), jnp.float32),
            pltpu.VMEM((G, EMB), jnp.float32),
            pltpu.VMEM((G, EMB), jnp.float32),
            pltpu.VMEM((G, EMB), jnp.float32),
            pltpu.VMEM_SHARED((NUM_LOC + NUM_AGE, EMB), jnp.float32),
            pltpu.SemaphoreType.DMA,
            pltpu.SemaphoreType.DMA,
            pltpu.SemaphoreType.DMA,
            pltpu.SemaphoreType.DMA,
        ],
    )
    def _sc_gather(loc_idx_hbm, age_idx_hbm, loc_tab_hbm, age_tab_hbm,
                   out_hbm, loc_v, age_v, lbuf0, lbuf1, abuf0, abuf1,
                   table_sh, gsem0, gsem1, wsem0, wsem1):
        lbufs = (lbuf0, lbuf1)
        abufs = (abuf0, abuf1)
        gsems = (gsem0, gsem1)
        wsems = (wsem0, wsem1)
        sid = lax.axis_index("s")
        wid = sid * NC + lax.axis_index("c")
        base = wid * CPW

        @pl.when(sid == 0)
        def _():
            pltpu.sync_copy(loc_tab_hbm, table_sh.at[pl.ds(0, NUM_LOC)])
            pltpu.sync_copy(age_tab_hbm,
                            table_sh.at[pl.ds(NUM_LOC, NUM_AGE)])

        cp_l = pltpu.async_copy(loc_idx_hbm.at[pl.ds(base, CPW)], loc_v, gsem0)
        cp_a = pltpu.async_copy(age_idx_hbm.at[pl.ds(base, CPW)], age_v, gsem1)
        cp_l.wait()
        cp_a.wait()

        plsc.subcore_barrier()

        age_tab_sh = table_sh.at[pl.ds(NUM_LOC, NUM_AGE)]

        def start_gathers(k, b):
            return [
                pltpu.async_copy(
                    table_sh.at[loc_v.at[pl.ds(k * G, G)]], lbufs[b], gsems[b]),
                pltpu.async_copy(
                    age_tab_sh.at[age_v.at[pl.ds(k * G, G)]], abufs[b],
                    gsems[b]),
            ]

        gd = [None] * NCHUNK
        wd = [None] * NCHUNK
        gd[0] = start_gathers(0, 0)
        for k in range(NCHUNK):
            b = k & 1
            for d in gd[k]:
                d.wait()
            if k + 1 < NCHUNK:
                if k >= 1:
                    for d in wd[k - 1]:
                        d.wait()
                gd[k + 1] = start_gathers(k + 1, 1 - b)
            row0 = base + k * G
            wd[k] = [
                pltpu.async_copy(
                    lbufs[b], out_hbm.at[pl.ds(row0, G), pl.ds(0, EMB)],
                    wsems[b]),
                pltpu.async_copy(
                    abufs[b], out_hbm.at[pl.ds(row0, G), pl.ds(EMB, EMB)],
                    wsems[b]),
            ]
        for k in (NCHUNK - 2, NCHUNK - 1):
            for d in wd[k]:
                d.wait()

    return _sc_gather


def kernel(location_idx, age_idx, location_table, age_table):
    return _build_sc_gather()(location_idx.astype(jnp.int32),
                              age_idx.astype(jnp.int32),
                              location_table, age_table)


# speedup vs baseline: 1.0139x; 1.0139x over previous
import functools

import jax
import jax.numpy as jnp
from jax import lax
from jax.experimental import pallas as pl
from jax.experimental.pallas import tpu as pltpu
from jax.experimental.pallas import tpu_sc as plsc

B = 16384
EMB = 128
NUM_LOC = 58
NUM_AGE = 2

NC, NS, L = 2, 16, 16
NW = NC * NS
CPW = B // NW
G = 128
NCHUNK = CPW // G


@functools.lru_cache(maxsize=None)
def _build_sc_gather():
    @functools.partial(
        pl.kernel,
        mesh=plsc.VectorSubcoreMesh(core_axis_name="c", subcore_axis_name="s"),
        compiler_params=pltpu.CompilerParams(
            needs_layout_passes=False,
            skip_device_barrier=True,
            disable_bounds_checks=True,
            disable_semaphore_checks=True,
        ),
        out_type=jax.ShapeDtypeStruct((B, 2 * EMB), jnp.float32),
        scratch_types=[
            pltpu.VMEM((CPW,), jnp.int32),
            pltpu.VMEM((CPW,), jnp.int32),
            pltpu.VMEM((G, EMB), jnp.float32),
            pltpu.VMEM((G, EMB), jnp.float32),
            pltpu.VMEM((G, EMB), jnp.float32),
            pltpu.VMEM((G, EMB), jnp.float32),
            pltpu.VMEM_SHARED((NUM_LOC + NUM_AGE, EMB), jnp.float32),
            pltpu.SemaphoreType.DMA,
            pltpu.SemaphoreType.DMA,
            pltpu.SemaphoreType.DMA,
            pltpu.SemaphoreType.DMA,
        ],
    )
    def _sc_gather(loc_idx_hbm, age_idx_hbm, loc_tab_hbm, age_tab_hbm,
                   out_hbm, loc_v, age_v, lbuf0, lbuf1, abuf0, abuf1,
                   table_sh, gsem0, gsem1, wsem0, wsem1):
        lbufs = (lbuf0, lbuf1)
        abufs = (abuf0, abuf1)
        gsems = (gsem0, gsem1)
        wsems = (wsem0, wsem1)
        sid = lax.axis_index("s")
        wid = sid * NC + lax.axis_index("c")
        base = wid * CPW

        @pl.when(sid == 0)
        def _():
            tp_l = pltpu.async_copy(
                loc_tab_hbm, table_sh.at[pl.ds(0, NUM_LOC)], wsem0)
            tp_a = pltpu.async_copy(
                age_tab_hbm, table_sh.at[pl.ds(NUM_LOC, NUM_AGE)], wsem1)
            tp_l.wait()
            tp_a.wait()

        cp_l = pltpu.async_copy(loc_idx_hbm.at[pl.ds(base, CPW)], loc_v, gsem0)
        cp_a = pltpu.async_copy(age_idx_hbm.at[pl.ds(base, CPW)], age_v, gsem1)
        cp_l.wait()
        cp_a.wait()

        plsc.subcore_barrier()

        age_tab_sh = table_sh.at[pl.ds(NUM_LOC, NUM_AGE)]

        def start_gathers(k, b):
            return [
                pltpu.async_copy(
                    table_sh.at[loc_v.at[pl.ds(k * G, G)]], lbufs[b], gsems[b]),
                pltpu.async_copy(
                    age_tab_sh.at[age_v.at[pl.ds(k * G, G)]], abufs[b],
                    gsems[b]),
            ]

        gd = [None] * NCHUNK
        wd = [None] * NCHUNK
        gd[0] = start_gathers(0, 0)
        for k in range(NCHUNK):
            b = k & 1
            for d in gd[k]:
                d.wait()
            if k + 1 < NCHUNK:
                if k >= 1:
                    for d in wd[k - 1]:
                        d.wait()
                gd[k + 1] = start_gathers(k + 1, 1 - b)
            row0 = base + k * G
            wd[k] = [
                pltpu.async_copy(
                    lbufs[b], out_hbm.at[pl.ds(row0, G), pl.ds(0, EMB)],
                    wsems[b]),
                pltpu.async_copy(
                    abufs[b], out_hbm.at[pl.ds(row0, G), pl.ds(EMB, EMB)],
                    wsems[b]),
            ]
        for k in (NCHUNK - 2, NCHUNK - 1):
            for d in wd[k]:
                d.wait()

    return _sc_gather


def kernel(location_idx, age_idx, location_table, age_table):
    return _build_sc_gather()(location_idx.astype(jnp.int32),
                              age_idx.astype(jnp.int32),
                              location_table, age_table)
